# Initial kernel scaffold; baseline (speedup 1.0000x reference)
import jax
import jax.numpy as jnp
from jax.experimental import pallas as pl


def kernel(x, edge_index, W_msg, b_msg, W_ih, W_hh, b_ih, b_hh):
    raise NotImplementedError("write your pallas kernel here")


# speedup vs baseline: 1.9300x; 1.9300x over previous
import functools

import jax
import jax.numpy as jnp
from jax import lax
from jax.experimental import pallas as pl
from jax.experimental.pallas import tpu as pltpu
from jax.experimental.pallas import tpu_sc as plsc

_CH = 128
_NW = 32


def _pre_body(h_ref, w0_ref, w1_ref, whh_ref, b0_ref, b1_ref, bhh_ref,
              a_ref, b_ref, gh_ref):
    h = h_ref[...]
    a_ref[...] = jnp.dot(h, w0_ref[...], preferred_element_type=jnp.float32) + b0_ref[...]
    b_ref[...] = jnp.dot(h, w1_ref[...], preferred_element_type=jnp.float32) + b1_ref[...]
    gh_ref[...] = jnp.dot(h, whh_ref[...], preferred_element_type=jnp.float32) + bhh_ref[...]


def _upd_body(p_ref, h_ref, gh_ref, wih_ref, bih_ref, out_ref):
    agg = p_ref[0] + p_ref[1]
    gi = jnp.dot(agg, wih_ref[...], preferred_element_type=jnp.float32) + bih_ref[...]
    gh = gh_ref[...]
    hdim = h_ref.shape[-1]
    r = jax.nn.sigmoid(gi[:, :hdim] + gh[:, :hdim])
    z = jax.nn.sigmoid(gi[:, hdim:2 * hdim] + gh[:, hdim:2 * hdim])
    n = jnp.tanh(gi[:, 2 * hdim:] + r * gh[:, 2 * hdim:])
    out_ref[...] = (1.0 - z) * n + z * h_ref[...]


def _make_pre(n, h, r):
    grid = n // r
    return pl.pallas_call(
        _pre_body,
        grid=(grid,),
        in_specs=[
            pl.BlockSpec((r, h), lambda i: (i, 0)),
            pl.BlockSpec((h, h), lambda i: (0, 0)),
            pl.BlockSpec((h, h), lambda i: (0, 0)),
            pl.BlockSpec((h, 3 * h), lambda i: (0, 0)),
            pl.BlockSpec((1, h), lambda i: (0, 0)),
            pl.BlockSpec((1, h), lambda i: (0, 0)),
            pl.BlockSpec((1, 3 * h), lambda i: (0, 0)),
        ],
        out_specs=[
            pl.BlockSpec((r, h), lambda i: (i, 0)),
            pl.BlockSpec((r, h), lambda i: (i, 0)),
            pl.BlockSpec((r, 3 * h), lambda i: (i, 0)),
        ],
        out_shape=[
            jax.ShapeDtypeStruct((n, h), jnp.float32),
            jax.ShapeDtypeStruct((n, h), jnp.float32),
            jax.ShapeDtypeStruct((n, 3 * h), jnp.float32),
        ],
    )


def _make_upd(n, h, r, n_pad):
    grid = n // r
    return pl.pallas_call(
        _upd_body,
        grid=(grid,),
        in_specs=[
            pl.BlockSpec((2, r, h), lambda i: (0, i, 0)),
            pl.BlockSpec((r, h), lambda i: (i, 0)),
            pl.BlockSpec((r, 3 * h), lambda i: (i, 0)),
            pl.BlockSpec((h, 3 * h), lambda i: (0, 0)),
            pl.BlockSpec((1, 3 * h), lambda i: (0, 0)),
        ],
        out_specs=pl.BlockSpec((r, h), lambda i: (i, 0)),
        out_shape=jax.ShapeDtypeStruct((n, h), jnp.float32),
    )


def _make_sc(n_pad, h, nch):
    mesh = plsc.VectorSubcoreMesh(core_axis_name="c", subcore_axis_name="s")
    rpt = n_pad // 16

    nlane = h // 16

    @functools.partial(
        pl.kernel,
        mesh=mesh,
        out_type=jax.ShapeDtypeStruct((2, n_pad, h), jnp.float32),
        scratch_types=[
            pltpu.VMEM_SHARED((n_pad, h), jnp.float32),
            pltpu.VMEM((_CH,), jnp.int32),
            pltpu.VMEM((_CH,), jnp.int32),
            pltpu.VMEM((_CH, 16), jnp.float32),
            pltpu.VMEM((_CH, h), jnp.float32),
            pltpu.VMEM((_CH, h), jnp.float32),
            pltpu.SemaphoreType.DMA,
        ],
    )
    def sc_agg(tab_hbm, g_hbm, f_hbm, m_hbm, zer_hbm, out_hbm,
               agg, idxg, idxf, msk, rows, racc, sem):
        cid = lax.axis_index("c")
        sid = lax.axis_index("s")
        wid = cid * 16 + sid

        pltpu.sync_copy(zer_hbm.at[pl.ds(sid * rpt, rpt)],
                        agg.at[pl.ds(sid * rpt, rpt)])

        plsc.subcore_barrier()

        def chunk(j, acc):
            pltpu.sync_copy(g_hbm.at[wid, j], idxg)
            cp = pltpu.async_copy(tab_hbm.at[idxg], rows, sem)
            pltpu.sync_copy(f_hbm.at[wid, j], idxf)
            pltpu.sync_copy(m_hbm.at[wid, j], msk)
            cp.wait()

            def edge(i, acc):
                m = msk[i]
                acc = tuple(
                    rows[i, 16 * k:16 * (k + 1)] + acc[k] * m
                    for k in range(nlane)
                )
                for k in range(nlane):
                    racc[i, 16 * k:16 * (k + 1)] = acc[k]
                return acc

            acc = lax.fori_loop(0, _CH, edge, acc)
            pltpu.sync_copy(racc, agg.at[idxf], add=True)
            return acc

        zacc = tuple(jnp.zeros((16,), jnp.float32) for _ in range(nlane))
        lax.fori_loop(0, nch, chunk, zacc)

        plsc.subcore_barrier()

        pltpu.sync_copy(agg.at[pl.ds(sid * rpt, rpt)],
                        out_hbm.at[cid, pl.ds(sid * rpt, rpt)])

    return sc_agg


def kernel(x, edge_index, W_msg, b_msg, W_ih, W_hh, b_ih, b_hh):
    n, h = x.shape
    e = edge_index.shape[1]
    t_steps = 5

    per_w = -(-e // _NW)
    ep = -(-per_w // _CH) * _CH
    e_pad = ep * _NW
    nch1 = ep // _CH
    nch = 2 * nch1
    n_pad = -(-(n + 1) // 128) * 128

    src = edge_index[0].astype(jnp.int32)
    dst = edge_index[1].astype(jnp.int32)

    def _build(tgt, gi):
        order = jnp.argsort(tgt, stable=True)
        pad = e_pad - tgt.shape[0]
        tgt_p = jnp.concatenate([tgt[order], jnp.full((pad,), n, jnp.int32)])
        gi_p = jnp.concatenate([gi[order], jnp.zeros((pad,), jnp.int32)])
        pos = jnp.arange(e_pad, dtype=jnp.int32)
        carry = (tgt_p == jnp.roll(tgt_p, 1)) & (pos % ep != 0)
        endr = (tgt_p != jnp.roll(tgt_p, -1)) | (pos % ep == ep - 1)
        fidx = jnp.where(endr, tgt_p, n)
        return (gi_p.reshape(_NW, nch1, _CH),
                fidx.reshape(_NW, nch1, _CH),
                jnp.broadcast_to(carry.astype(jnp.float32)[:, None],
                                 (e_pad, 16)).reshape(_NW, nch1, _CH, 16))

    g0, f0, m0 = _build(dst, src)
    g1, f1, m1 = _build(src, dst + n)
    g_arr = jnp.concatenate([g0, g1], axis=1)
    f_arr = jnp.concatenate([f0, f1], axis=1)
    m_arr = jnp.concatenate([m0, m1], axis=1)
    zer = jnp.zeros((n_pad, h), jnp.float32)

    w0 = W_msg[0]
    w1 = W_msg[1]
    b0 = b_msg[0][None, :]
    b1 = b_msg[1][None, :]
    wih_t = W_ih.T
    whh_t = W_hh.T
    bih = b_ih[None, :]
    bhh = b_hh[None, :]

    r = 1000 if n % 1000 == 0 else (500 if n % 500 == 0 else 8)
    pre = _make_pre(n, h, r)
    upd = _make_upd(n, h, r, n_pad)
    sc_agg = _make_sc(n_pad, h, nch)

    hs = x
    a, b, gh = pre(hs, w0, w1, whh_t, b0, b1, bhh)
    for t in range(t_steps):
        tab = jnp.concatenate([a, b], axis=0)
        partials = sc_agg(tab, g_arr, f_arr, m_arr, zer)
        hs = upd(partials, hs, gh, wih_t, bih)
        if t + 1 < t_steps:
            a, b, gh = pre(hs, w0, w1, whh_t, b0, b1, bhh)
    return hs


# speedup vs baseline: 2.1391x; 1.1083x over previous
import functools

import jax
import jax.numpy as jnp
from jax import lax
from jax.experimental import pallas as pl
from jax.experimental.pallas import tpu as pltpu
from jax.experimental.pallas import tpu_sc as plsc

_CH = 80
_NW = 32


def _pre_body(h_ref, w0_ref, w1_ref, whh_ref, b0_ref, b1_ref, bhh_ref,
              a_ref, b_ref, gh_ref):
    h = h_ref[...]
    a_ref[...] = jnp.dot(h, w0_ref[...], preferred_element_type=jnp.float32) + b0_ref[...]
    b_ref[...] = jnp.dot(h, w1_ref[...], preferred_element_type=jnp.float32) + b1_ref[...]
    gh_ref[...] = jnp.dot(h, whh_ref[...], preferred_element_type=jnp.float32) + bhh_ref[...]


def _upd_body(p_ref, h_ref, gh_ref, wih_ref, bih_ref, out_ref):
    agg = p_ref[0] + p_ref[1]
    gi = jnp.dot(agg, wih_ref[...], preferred_element_type=jnp.float32) + bih_ref[...]
    gh = gh_ref[...]
    hdim = h_ref.shape[-1]
    r = jax.nn.sigmoid(gi[:, :hdim] + gh[:, :hdim])
    z = jax.nn.sigmoid(gi[:, hdim:2 * hdim] + gh[:, hdim:2 * hdim])
    n = jnp.tanh(gi[:, 2 * hdim:] + r * gh[:, 2 * hdim:])
    out_ref[...] = (1.0 - z) * n + z * h_ref[...]


def _make_pre(n, h, r):
    grid = n // r
    return pl.pallas_call(
        _pre_body,
        grid=(grid,),
        in_specs=[
            pl.BlockSpec((r, h), lambda i: (i, 0)),
            pl.BlockSpec((h, h), lambda i: (0, 0)),
            pl.BlockSpec((h, h), lambda i: (0, 0)),
            pl.BlockSpec((h, 3 * h), lambda i: (0, 0)),
            pl.BlockSpec((1, h), lambda i: (0, 0)),
            pl.BlockSpec((1, h), lambda i: (0, 0)),
            pl.BlockSpec((1, 3 * h), lambda i: (0, 0)),
        ],
        out_specs=[
            pl.BlockSpec((r, h), lambda i: (i, 0)),
            pl.BlockSpec((r, h), lambda i: (i, 0)),
            pl.BlockSpec((r, 3 * h), lambda i: (i, 0)),
        ],
        out_shape=[
            jax.ShapeDtypeStruct((n, h), jnp.float32),
            jax.ShapeDtypeStruct((n, h), jnp.float32),
            jax.ShapeDtypeStruct((n, 3 * h), jnp.float32),
        ],
    )


def _make_upd(n, h, r, n_pad):
    grid = n // r
    return pl.pallas_call(
        _upd_body,
        grid=(grid,),
        in_specs=[
            pl.BlockSpec((2, r, h), lambda i: (0, i, 0)),
            pl.BlockSpec((r, h), lambda i: (i, 0)),
            pl.BlockSpec((r, 3 * h), lambda i: (i, 0)),
            pl.BlockSpec((h, 3 * h), lambda i: (0, 0)),
            pl.BlockSpec((1, 3 * h), lambda i: (0, 0)),
        ],
        out_specs=pl.BlockSpec((r, h), lambda i: (i, 0)),
        out_shape=jax.ShapeDtypeStruct((n, h), jnp.float32),
    )


def _make_sc(n_pad, h, nch):
    mesh = plsc.VectorSubcoreMesh(core_axis_name="c", subcore_axis_name="s")
    rpt = n_pad // 16

    nlane = h // 16

    @functools.partial(
        pl.kernel,
        mesh=mesh,
        out_type=jax.ShapeDtypeStruct((2, n_pad, h), jnp.float32),
        scratch_types=[
            pltpu.VMEM_SHARED((n_pad, h), jnp.float32),
            pltpu.VMEM((_CH,), jnp.int32),
            pltpu.VMEM((_CH,), jnp.int32),
            pltpu.VMEM((_CH,), jnp.int32),
            pltpu.VMEM((_CH, 16), jnp.float32),
            pltpu.VMEM((_CH, h), jnp.float32),
            pltpu.VMEM((_CH, h), jnp.float32),
            pltpu.VMEM((_CH, h), jnp.float32),
            pltpu.SemaphoreType.DMA,
            pltpu.SemaphoreType.DMA,
        ],
    )
    def sc_agg(tab_hbm, g_hbm, f_hbm, m_hbm, zer_hbm, out_hbm,
               agg, ig0, ig1, idxf, msk, rows0, rows1, racc, sem0, sem1):
        cid = lax.axis_index("c")
        sid = lax.axis_index("s")
        wid = cid * 16 + sid

        pltpu.sync_copy(zer_hbm.at[pl.ds(sid * rpt, rpt)],
                        agg.at[pl.ds(sid * rpt, rpt)])

        plsc.subcore_barrier()

        pltpu.sync_copy(g_hbm.at[wid, 0], ig0)
        pltpu.make_async_copy(tab_hbm.at[ig0], rows0, sem0).start()
        pltpu.sync_copy(g_hbm.at[wid, 1], ig1)
        pltpu.make_async_copy(tab_hbm.at[ig1], rows1, sem1).start()

        def do_chunk(j, acc, ig, rows, sem):
            pltpu.make_async_copy(tab_hbm.at[ig], rows, sem).wait()
            pltpu.sync_copy(f_hbm.at[wid, j], idxf)
            pltpu.sync_copy(m_hbm.at[wid, j], msk)

            def edge(i, acc):
                m = msk[i]
                acc = tuple(
                    rows[i, 16 * k:16 * (k + 1)] + acc[k] * m
                    for k in range(nlane)
                )
                for k in range(nlane):
                    racc[i, 16 * k:16 * (k + 1)] = acc[k]
                return acc

            acc = lax.fori_loop(0, _CH, edge, acc, unroll=8)

            @pl.when(j + 2 < nch)
            def _():
                pltpu.sync_copy(g_hbm.at[wid, j + 2], ig)
                pltpu.make_async_copy(tab_hbm.at[ig], rows, sem).start()

            pltpu.sync_copy(racc, agg.at[idxf], add=True)
            return acc

        def pair(jj, acc):
            acc = do_chunk(2 * jj, acc, ig0, rows0, sem0)
            acc = do_chunk(2 * jj + 1, acc, ig1, rows1, sem1)
            return acc

        zacc = tuple(jnp.zeros((16,), jnp.float32) for _ in range(nlane))
        lax.fori_loop(0, nch // 2, pair, zacc)

        plsc.subcore_barrier()

        pltpu.sync_copy(agg.at[pl.ds(sid * rpt, rpt)],
                        out_hbm.at[cid, pl.ds(sid * rpt, rpt)])

    return sc_agg


def kernel(x, edge_index, W_msg, b_msg, W_ih, W_hh, b_ih, b_hh):
    n, h = x.shape
    e = edge_index.shape[1]
    t_steps = 5

    per_w = -(-e // _NW)
    ep = -(-per_w // _CH) * _CH
    e_pad = ep * _NW
    nch1 = ep // _CH
    nch = 2 * nch1
    n_pad = -(-(n + 1) // 128) * 128

    src = edge_index[0].astype(jnp.int32)
    dst = edge_index[1].astype(jnp.int32)

    def _build(tgt, gi):
        order = jnp.argsort(tgt, stable=True)
        pad = e_pad - tgt.shape[0]
        tgt_p = jnp.concatenate([tgt[order], jnp.full((pad,), n, jnp.int32)])
        gi_p = jnp.concatenate([gi[order], jnp.zeros((pad,), jnp.int32)])
        pos = jnp.arange(e_pad, dtype=jnp.int32)
        carry = (tgt_p == jnp.roll(tgt_p, 1)) & (pos % ep != 0)
        endr = (tgt_p != jnp.roll(tgt_p, -1)) | (pos % ep == ep - 1)
        fidx = jnp.where(endr, tgt_p, n)
        return (gi_p.reshape(_NW, nch1, _CH),
                fidx.reshape(_NW, nch1, _CH),
                jnp.broadcast_to(carry.astype(jnp.float32)[:, None],
                                 (e_pad, 16)).reshape(_NW, nch1, _CH, 16))

    g0, f0, m0 = _build(dst, src)
    g1, f1, m1 = _build(src, dst + n)
    g_arr = jnp.concatenate([g0, g1], axis=1)
    f_arr = jnp.concatenate([f0, f1], axis=1)
    m_arr = jnp.concatenate([m0, m1], axis=1)
    zer = jnp.zeros((n_pad, h), jnp.float32)

    w0 = W_msg[0]
    w1 = W_msg[1]
    b0 = b_msg[0][None, :]
    b1 = b_msg[1][None, :]
    wih_t = W_ih.T
    whh_t = W_hh.T
    bih = b_ih[None, :]
    bhh = b_hh[None, :]

    r = 1000 if n % 1000 == 0 else (500 if n % 500 == 0 else 8)
    pre = _make_pre(n, h, r)
    upd = _make_upd(n, h, r, n_pad)
    sc_agg = _make_sc(n_pad, h, nch)

    hs = x
    a, b, gh = pre(hs, w0, w1, whh_t, b0, b1, bhh)
    for t in range(t_steps):
        tab = jnp.concatenate([a, b], axis=0)
        partials = sc_agg(tab, g_arr, f_arr, m_arr, zer)
        hs = upd(partials, hs, gh, wih_t, bih)
        if t + 1 < t_steps:
            a, b, gh = pre(hs, w0, w1, whh_t, b0, b1, bhh)
    return hs
